# baseline (speedup 1.0000x reference)
import functools
import math

import numpy as np
import jax
import jax.numpy as jnp
from jax import lax
from jax.experimental import pallas as pl
from jax.experimental.pallas import tpu as pltpu
from jax.experimental.pallas import tpu_sc as plsc

_EPS = 1e-12


def _sin_tables(s, h):
    pos = np.arange(s, dtype=np.float32)[:, None]
    pe = np.zeros((s, h), np.float32)
    div = np.exp(np.arange(0, h, 2, dtype=np.float32) * -(math.log(10000.0) / h))
    pe[:, 0::2] = np.sin(pos * div)
    pe[:, 1::2] = np.cos(pos * div)
    hdim = h // 3
    hs = np.zeros((s, hdim), np.float32)
    divh = np.exp(np.arange(0, hdim, 2, dtype=np.float32) * -(math.log(10000.0) / hdim))
    hs[:, 0::2] = np.sin(pos * divh)
    hs[:, 1::2] = np.cos(pos * divh)
    return pe, hs


@functools.lru_cache(maxsize=None)
def _make_sc_kernel(B, S, H, C):
    info = plsc.get_sparse_core_info()
    NC, NS, L = info.num_cores, info.num_subcores, info.num_lanes
    NW = NC * NS
    T = B * S
    TPW = T // NW
    NCH = TPW // C
    NV = H // L
    HNV = (H // 3) // L
    WPB = NW // B
    assert T % NW == 0 and TPW % C == 0 and H % (3 * L) == 0 and NW % B == 0

    mesh = plsc.VectorSubcoreMesh(core_axis_name="c", subcore_axis_name="s")

    @functools.partial(
        pl.kernel,
        mesh=mesh,
        out_type=jax.ShapeDtypeStruct((T, H), jnp.float32),
        scratch_types=[
            pltpu.VMEM((TPW,), jnp.int32),
            pltpu.VMEM((TPW,), jnp.int32),
            pltpu.VMEM((TPW,), jnp.int32),
            pltpu.VMEM((2, H), jnp.float32),
            pltpu.VMEM((H,), jnp.float32),
            pltpu.VMEM((H,), jnp.float32),
            pltpu.VMEM((C, H), jnp.float32),
            pltpu.VMEM((C, H), jnp.float32),
            pltpu.VMEM((C, H // 3), jnp.float32),
            pltpu.SemaphoreType.DMA,
            pltpu.SemaphoreType.DMA,
        ],
    )
    def k(ids_h, para_h, tt_h, wtab_h, ttab_h, pe_h, hs_h, gam_h, bet_h, out_h,
          ids_v, para_v, tt_v, ttab_v, gam_v, bet_v, wbuf, pbuf, hbuf, sem, sem2):
        wid = lax.axis_index("s") * NC + lax.axis_index("c")
        t0 = wid * TPW
        s0 = (wid % WPB) * TPW
        pltpu.sync_copy(ids_h.at[pl.ds(t0, TPW)], ids_v)
        pltpu.sync_copy(para_h.at[pl.ds(t0, TPW)], para_v)
        pltpu.sync_copy(tt_h.at[pl.ds(t0, TPW)], tt_v)
        pltpu.sync_copy(ttab_h, ttab_v)
        pltpu.sync_copy(gam_h, gam_v)
        pltpu.sync_copy(bet_h, bet_v)

        def chunk_body(c, carry):
            cw = pltpu.async_copy(wtab_h.at[ids_v.at[pl.ds(c * C, C)]], wbuf, sem)
            ch = pltpu.async_copy(hs_h.at[para_v.at[pl.ds(c * C, C)]], hbuf, sem2)
            pltpu.sync_copy(pe_h.at[pl.ds(s0 + c * C, C)], pbuf)
            cw.wait()
            ch.wait()

            def tok_body(j, carry2):
                base = (j // L) * L
                off = j - base
                tvec = tt_v[pl.ds(c * C + base, L)]
                tfv = tvec.astype(jnp.float32)
                tf = tfv[jnp.zeros((L,), jnp.int32) + off]
                svec = jnp.zeros((L,), jnp.float32)
                qvec = jnp.zeros((L,), jnp.float32)
                for v in range(NV):
                    sl = pl.ds(v * L, L)
                    acc = (wbuf[j, sl] + pbuf[j, sl]
                           + hbuf[j, pl.ds((v % HNV) * L, L)]
                           + (ttab_v[0, sl] + tf * (ttab_v[1, sl] - ttab_v[0, sl])))
                    svec = svec + acc
                    qvec = qvec + acc * acc
                    wbuf[j, sl] = acc
                def lanesum(x):
                    for stride in (8, 4, 2, 1):
                        perm = lax.iota(jnp.int32, L) ^ stride
                        x = x + x[perm]
                    return x

                mv = lanesum(svec) * (1.0 / H)
                xv = lanesum(qvec) * (1.0 / H) - mv * mv + _EPS
                iv = lax.bitcast_convert_type(xv, jnp.int32)
                iv = jnp.int32(0x5F3759DF) - lax.shift_right_logical(
                    iv, jnp.full((L,), 1, jnp.int32))
                yv = lax.bitcast_convert_type(iv, jnp.float32)
                yv = yv * (1.5 - 0.5 * xv * yv * yv)
                yv = yv * (1.5 - 0.5 * xv * yv * yv)
                for v in range(NV):
                    sl = pl.ds(v * L, L)
                    wbuf[j, sl] = (wbuf[j, sl] - mv) * yv * gam_v[sl] + bet_v[sl]
                return carry2

            lax.fori_loop(0, C, tok_body, 0)
            pltpu.sync_copy(wbuf, out_h.at[pl.ds(t0 + c * C, C)])
            return carry

        lax.fori_loop(0, NCH, chunk_body, 0)

    return k


def kernel(input_ids, tok_struct_vec, sent_struct_vec, token_type_ids,
           word_table, type_table, ln_gamma, ln_beta):
    B, S = input_ids.shape
    H = word_table.shape[1]
    pe_np, hs_np = _sin_tables(S, H)
    ids = input_ids.reshape(-1).astype(jnp.int32)
    para = tok_struct_vec[..., 0].reshape(-1).astype(jnp.int32)
    tt = token_type_ids.reshape(-1).astype(jnp.int32)
    k = _make_sc_kernel(B, S, H, 32)
    out = k(ids, para, tt, word_table.astype(jnp.float32),
            type_table.astype(jnp.float32), jnp.asarray(pe_np),
            jnp.asarray(hs_np), ln_gamma.astype(jnp.float32),
            ln_beta.astype(jnp.float32))
    return out.reshape(B, S, H)
